# Initial kernel scaffold; baseline (speedup 1.0000x reference)
import jax
import jax.numpy as jnp
from jax import lax
from jax.experimental import pallas as pl
from jax.experimental.pallas import tpu as pltpu

N_UNITS = 512
INPUT_SIZE = 128
OUTPUT_SIZE = 128
E_IN = 4096
E_REC = 16384
M_GRID = 16
B = 16
T = 512
N_TOTAL = E_IN + E_REC + N_UNITS
CHUNK = 1024


def _prep_body(logits_t_ref, grid_t_ref, cl_t_ref, in_src_ref, in_dst_ref,
               rec_src_ref, rec_dst_ref, w_in_ref, w_rec_ref, bias_ref, ecl_ref):
    lt = logits_t_ref[...]
    m = jnp.max(lt, axis=0, keepdims=True)
    ii = lax.broadcasted_iota(jnp.int32, (M_GRID, N_TOTAL), 0)
    cand = jnp.where(lt == m, ii, M_GRID)
    mi = jnp.min(cand, axis=0, keepdims=True)
    onehot = (ii == mi).astype(jnp.float32)
    g = grid_t_ref[...]
    w = lax.dot_general(g, onehot, (((0,), (0,)), ((), ())),
                        preferred_element_type=jnp.float32)

    e = jnp.exp(lt - m)
    s = jnp.sum(e, axis=0, keepdims=True)
    cl = cl_t_ref[...]
    num = lax.dot_general(cl, e, (((0,), (0,)), ((), ())),
                          preferred_element_type=jnp.float32)
    ecl_ref[0, 0] = jnp.sum(num / s)

    bias_ref[...] = w[:, E_IN + E_REC:]

    acc_in = jnp.zeros((INPUT_SIZE, N_UNITS), jnp.float32)
    for c in range(E_IN // CHUNK):
        sl = slice(c * CHUNK, (c + 1) * CHUNK)
        srcv = in_src_ref[0:1, sl]
        dstv = in_dst_ref[0:1, sl]
        wv = w[:, c * CHUNK:(c + 1) * CHUNK]
        a = (lax.broadcasted_iota(jnp.int32, (INPUT_SIZE, CHUNK), 0)
             == srcv).astype(jnp.float32)
        bt = jnp.where(
            lax.broadcasted_iota(jnp.int32, (N_UNITS, CHUNK), 0) == dstv,
            wv, 0.0)
        acc_in = acc_in + lax.dot_general(
            a, bt, (((1,), (1,)), ((), ())), preferred_element_type=jnp.float32)
    w_in_ref[...] = acc_in

    acc = jnp.zeros((N_UNITS, N_UNITS), jnp.float32)
    for c in range(E_REC // CHUNK):
        sl = slice(c * CHUNK, (c + 1) * CHUNK)
        srcv = rec_src_ref[0:1, sl]
        dstv = rec_dst_ref[0:1, sl]
        wv = w[:, E_IN + c * CHUNK:E_IN + (c + 1) * CHUNK]
        a = (lax.broadcasted_iota(jnp.int32, (N_UNITS, CHUNK), 0)
             == srcv).astype(jnp.float32)
        bt = jnp.where(
            lax.broadcasted_iota(jnp.int32, (N_UNITS, CHUNK), 0) == dstv,
            wv, 0.0)
        acc = acc + lax.dot_general(
            a, bt, (((1,), (1,)), ((), ())), preferred_element_type=jnp.float32)
    w_rec_ref[...] = acc


def _scan_body(x_ref, w_in_ref, w_rec_ref, bias_ref, out_ref, h_ref):
    t = pl.program_id(0)

    @pl.when(t == 0)
    def _():
        h_ref[...] = jnp.zeros((B, N_UNITS), jnp.float32)

    xt = x_ref[0]
    oht = (lax.broadcasted_iota(jnp.int32, (INPUT_SIZE, B), 0)
           == xt).astype(jnp.float32)
    inc = lax.dot_general(oht, w_in_ref[...], (((0,), (0,)), ((), ())),
                          preferred_element_type=jnp.float32)
    rec = lax.dot_general(h_ref[...], w_rec_ref[...],
                          (((1,), (0,)), ((), ())),
                          preferred_element_type=jnp.float32)
    pre = bias_ref[...] + inc + rec
    mask = lax.broadcasted_iota(jnp.int32, (B, N_UNITS), 1) >= (
        N_UNITS - OUTPUT_SIZE)
    h_ref[...] = jnp.where(mask, pre, jnp.tanh(pre))
    out_ref[0] = pre[:, N_UNITS - OUTPUT_SIZE:]


def kernel(x, tau, logits, grid, codelengths, in_src, in_dst, rec_src,
           rec_dst, out_idx):
    del tau, out_idx
    logits_t = logits.T
    grid_t = grid.reshape(M_GRID, 1)
    cl_t = codelengths.reshape(M_GRID, 1)

    w_in, w_rec, bias, ecl = pl.pallas_call(
        _prep_body,
        out_shape=(
            jax.ShapeDtypeStruct((INPUT_SIZE, N_UNITS), jnp.float32),
            jax.ShapeDtypeStruct((N_UNITS, N_UNITS), jnp.float32),
            jax.ShapeDtypeStruct((1, N_UNITS), jnp.float32),
            jax.ShapeDtypeStruct((1, 1), jnp.float32),
        ),
    )(logits_t, grid_t, cl_t,
      in_src.reshape(1, E_IN), in_dst.reshape(1, E_IN),
      rec_src.reshape(1, E_REC), rec_dst.reshape(1, E_REC))

    x3 = x.T.reshape(T, 1, B)
    out_t = pl.pallas_call(
        _scan_body,
        grid=(T,),
        in_specs=[
            pl.BlockSpec((1, 1, B), lambda t: (t, 0, 0)),
            pl.BlockSpec((INPUT_SIZE, N_UNITS), lambda t: (0, 0)),
            pl.BlockSpec((N_UNITS, N_UNITS), lambda t: (0, 0)),
            pl.BlockSpec((1, N_UNITS), lambda t: (0, 0)),
        ],
        out_specs=pl.BlockSpec((1, B, OUTPUT_SIZE), lambda t: (t, 0, 0)),
        out_shape=jax.ShapeDtypeStruct((T, B, OUTPUT_SIZE), jnp.float32),
        scratch_shapes=[pltpu.VMEM((B, N_UNITS), jnp.float32)],
    )(x3, w_in, w_rec, bias)

    logits_out = jnp.transpose(out_t, (1, 0, 2))
    return logits_out, ecl[0, 0]


# speedup vs baseline: 326.2790x; 326.2790x over previous
import jax
import jax.numpy as jnp
from jax import lax
from jax.experimental import pallas as pl
from jax.experimental.pallas import tpu as pltpu

N_UNITS = 512
INPUT_SIZE = 128
OUTPUT_SIZE = 128
E_IN = 4096
E_REC = 16384
M_GRID = 16
B = 16
T = 512
N_TOTAL = E_IN + E_REC + N_UNITS
CHUNK = 1024


def _prep_body(logits_t_ref, grid_t_ref, cl_t_ref, in_src_ref, in_dst_ref,
               rec_src_ref, rec_dst_ref, w_in_ref, w_rec_ref, bias_ref, ecl_ref):
    lt = logits_t_ref[...]
    m = jnp.max(lt, axis=0, keepdims=True)
    ii = lax.broadcasted_iota(jnp.int32, (M_GRID, N_TOTAL), 0)
    cand = jnp.where(lt == m, ii, M_GRID)
    mi = jnp.min(cand, axis=0, keepdims=True)
    onehot = (ii == mi).astype(jnp.float32)
    g = grid_t_ref[...]
    w = lax.dot_general(g, onehot, (((0,), (0,)), ((), ())),
                        preferred_element_type=jnp.float32)

    e = jnp.exp(lt - m)
    s = jnp.sum(e, axis=0, keepdims=True)
    cl = cl_t_ref[...]
    num = lax.dot_general(cl, e, (((0,), (0,)), ((), ())),
                          preferred_element_type=jnp.float32)
    ecl_ref[...] = jnp.sum(num / s, axis=1, keepdims=True)

    bias_ref[...] = w[:, E_IN + E_REC:]

    acc_in = jnp.zeros((INPUT_SIZE, N_UNITS), jnp.float32)
    for c in range(E_IN // CHUNK):
        sl = slice(c * CHUNK, (c + 1) * CHUNK)
        srcv = in_src_ref[0:1, sl]
        dstv = in_dst_ref[0:1, sl]
        wv = w[:, c * CHUNK:(c + 1) * CHUNK]
        a = (lax.broadcasted_iota(jnp.int32, (INPUT_SIZE, CHUNK), 0)
             == srcv).astype(jnp.float32)
        bt = jnp.where(
            lax.broadcasted_iota(jnp.int32, (N_UNITS, CHUNK), 0) == dstv,
            wv, 0.0)
        acc_in = acc_in + lax.dot_general(
            a, bt, (((1,), (1,)), ((), ())), preferred_element_type=jnp.float32)
    w_in_ref[...] = acc_in

    acc = jnp.zeros((N_UNITS, N_UNITS), jnp.float32)
    for c in range(E_REC // CHUNK):
        sl = slice(c * CHUNK, (c + 1) * CHUNK)
        srcv = rec_src_ref[0:1, sl]
        dstv = rec_dst_ref[0:1, sl]
        wv = w[:, E_IN + c * CHUNK:E_IN + (c + 1) * CHUNK]
        a = (lax.broadcasted_iota(jnp.int32, (N_UNITS, CHUNK), 0)
             == srcv).astype(jnp.float32)
        bt = jnp.where(
            lax.broadcasted_iota(jnp.int32, (N_UNITS, CHUNK), 0) == dstv,
            wv, 0.0)
        acc = acc + lax.dot_general(
            a, bt, (((1,), (1,)), ((), ())), preferred_element_type=jnp.float32)
    w_rec_ref[...] = acc


def _scan_body(x_ref, w_in_ref, w_rec_ref, bias_ref, out_ref, h_ref):
    t = pl.program_id(0)

    @pl.when(t == 0)
    def _():
        h_ref[...] = jnp.zeros((B, N_UNITS), jnp.float32)

    xt = x_ref[0]
    oht = (lax.broadcasted_iota(jnp.int32, (INPUT_SIZE, B), 0)
           == xt).astype(jnp.float32)
    inc = lax.dot_general(oht, w_in_ref[...], (((0,), (0,)), ((), ())),
                          preferred_element_type=jnp.float32)
    rec = lax.dot_general(h_ref[...], w_rec_ref[...],
                          (((1,), (0,)), ((), ())),
                          preferred_element_type=jnp.float32)
    pre = bias_ref[...] + inc + rec
    mask = lax.broadcasted_iota(jnp.int32, (B, N_UNITS), 1) >= (
        N_UNITS - OUTPUT_SIZE)
    h_ref[...] = jnp.where(mask, pre, jnp.tanh(pre))
    out_ref[0] = pre[:, N_UNITS - OUTPUT_SIZE:]


def kernel(x, tau, logits, grid, codelengths, in_src, in_dst, rec_src,
           rec_dst, out_idx):
    del tau, out_idx
    logits_t = logits.T
    grid_t = grid.reshape(M_GRID, 1)
    cl_t = codelengths.reshape(M_GRID, 1)

    w_in, w_rec, bias, ecl = pl.pallas_call(
        _prep_body,
        out_shape=(
            jax.ShapeDtypeStruct((INPUT_SIZE, N_UNITS), jnp.float32),
            jax.ShapeDtypeStruct((N_UNITS, N_UNITS), jnp.float32),
            jax.ShapeDtypeStruct((1, N_UNITS), jnp.float32),
            jax.ShapeDtypeStruct((1, 1), jnp.float32),
        ),
    )(logits_t, grid_t, cl_t,
      in_src.reshape(1, E_IN), in_dst.reshape(1, E_IN),
      rec_src.reshape(1, E_REC), rec_dst.reshape(1, E_REC))

    x3 = x.T.reshape(T, 1, B)
    out_t = pl.pallas_call(
        _scan_body,
        grid=(T,),
        in_specs=[
            pl.BlockSpec((1, 1, B), lambda t: (t, 0, 0)),
            pl.BlockSpec((INPUT_SIZE, N_UNITS), lambda t: (0, 0)),
            pl.BlockSpec((N_UNITS, N_UNITS), lambda t: (0, 0)),
            pl.BlockSpec((1, N_UNITS), lambda t: (0, 0)),
        ],
        out_specs=pl.BlockSpec((1, B, OUTPUT_SIZE), lambda t: (t, 0, 0)),
        out_shape=jax.ShapeDtypeStruct((T, B, OUTPUT_SIZE), jnp.float32),
        scratch_shapes=[pltpu.VMEM((B, N_UNITS), jnp.float32)],
    )(x3, w_in, w_rec, bias)

    logits_out = jnp.transpose(out_t, (1, 0, 2))
    return logits_out, ecl[0, 0]


# speedup vs baseline: 340.5364x; 1.0437x over previous
import functools

import jax
import jax.numpy as jnp
from jax import lax
from jax.experimental import pallas as pl
from jax.experimental.pallas import tpu as pltpu
from jax.experimental.pallas import tpu_sc as plsc

N_UNITS = 512
INPUT_SIZE = 128
OUTPUT_SIZE = 128
E_IN = 4096
E_REC = 16384
M_GRID = 16
B = 16
T = 512
N_TOTAL = E_IN + E_REC + N_UNITS

NW = 32
ROWS_W = N_TOTAL // NW
BLOCKS_W = ROWS_W // 16
PAD_W = 768
W_REC_OFF = INPUT_SIZE * N_UNITS
BIAS_OFF = W_REC_OFF + N_UNITS * N_UNITS
TRASH_OFF = BIAS_OFF + N_UNITS
DENSE_ROWS = 642
DENSE = DENSE_ROWS * N_UNITS
ZCHUNK = DENSE // 16


def _sc_prep_body(logits_hbm, grid_hbm, cl_hbm, idx_hbm, dense_hbm, ecl_hbm,
                  lbuf, wbuf, idxbuf, gridbuf, clbuf, eclbuf, zbuf, shared):
    cid = lax.axis_index("c")
    sid = lax.axis_index("s")
    w = sid * 2 + cid

    pltpu.sync_copy(logits_hbm.at[pl.ds(w * ROWS_W * M_GRID, ROWS_W * M_GRID)],
                    lbuf)
    pltpu.sync_copy(grid_hbm, gridbuf)
    pltpu.sync_copy(cl_hbm, clbuf)
    pltpu.sync_copy(idx_hbm.at[w], idxbuf)

    def zero_body(i, carry):
        zbuf[pl.ds(i * 16, 16)] = jnp.zeros((16,), jnp.float32)
        return carry
    lax.fori_loop(0, ZCHUNK // 16, zero_body, 0)
    pltpu.sync_copy(zbuf, shared.at[pl.ds(sid * ZCHUNK, ZCHUNK)])

    iota = lax.broadcasted_iota(jnp.int32, (16,), 0)
    row_stride = iota * M_GRID
    clv = clbuf[...]

    def block_body(b, ecl_acc):
        base = b * (16 * M_GRID)
        mv = jnp.full((16,), -jnp.inf, jnp.float32)
        ai = jnp.zeros((16,), jnp.int32)
        cols = []
        for m in range(M_GRID):
            v = plsc.load_gather(lbuf, [base + row_stride + m])
            upd = v > mv
            mv = jnp.where(upd, v, mv)
            ai = jnp.where(upd, m, ai)
            cols.append(v)
        wv = plsc.load_gather(gridbuf, [ai])
        wbuf[pl.ds(b * 16, 16)] = wv
        sv = jnp.zeros((16,), jnp.float32)
        nv = jnp.zeros((16,), jnp.float32)
        for m in range(M_GRID):
            e = jnp.exp(cols[m] - mv)
            sv = sv + e
            nv = nv + clv[m] * e
        return ecl_acc + nv / sv

    ecl_acc = lax.fori_loop(0, BLOCKS_W, block_body,
                            jnp.zeros((16,), jnp.float32))

    for j in range(BLOCKS_W, PAD_W // 16):
        wbuf[pl.ds(j * 16, 16)] = jnp.zeros((16,), jnp.float32)

    plsc.subcore_barrier()
    for j in range(PAD_W // 128):
        pltpu.sync_copy(wbuf.at[pl.ds(j * 128, 128)],
                        shared.at[idxbuf.at[j]], add=True)
    plsc.subcore_barrier()

    @pl.when(sid == 0)
    def _():
        pltpu.sync_copy(shared, dense_hbm.at[cid])

    tot = jnp.sum(ecl_acc)
    eclbuf[...] = jnp.where(iota == 0, tot, 0.0)
    pltpu.sync_copy(eclbuf, ecl_hbm.at[w])


_sc_prep = functools.partial(
    pl.kernel,
    mesh=plsc.VectorSubcoreMesh(core_axis_name="c", subcore_axis_name="s"),
    compiler_params=pltpu.CompilerParams(needs_layout_passes=False),
    out_type=(
        jax.ShapeDtypeStruct((2, DENSE), jnp.float32),
        jax.ShapeDtypeStruct((NW, 16), jnp.float32),
    ),
    scratch_types=[
        pltpu.VMEM((ROWS_W * M_GRID,), jnp.float32),
        pltpu.VMEM((PAD_W,), jnp.float32),
        pltpu.VMEM((PAD_W // 128, 128), jnp.int32),
        pltpu.VMEM((16,), jnp.float32),
        pltpu.VMEM((16,), jnp.float32),
        pltpu.VMEM((16,), jnp.float32),
        pltpu.VMEM((ZCHUNK,), jnp.float32),
        pltpu.VMEM_SHARED((DENSE,), jnp.float32),
    ],
)(_sc_prep_body)


def _scan_body(x_ref, dwin_ref, dwrec_ref, dbias_ref, eclp_ref,
               out_ref, ecl_ref):
    win = dwin_ref[0] + dwin_ref[1]
    wrec = dwrec_ref[0] + dwrec_ref[1]
    bias = dbias_ref[0] + dbias_ref[1]
    p = jnp.sum(eclp_ref[...], axis=0, keepdims=True)
    ecl_ref[...] = jnp.sum(p, axis=1, keepdims=True)
    mask = lax.broadcasted_iota(jnp.int32, (B, N_UNITS), 1) >= (
        N_UNITS - OUTPUT_SIZE)

    win_hi = win.astype(jnp.bfloat16)
    win_lo = (win - win_hi.astype(jnp.float32)).astype(jnp.bfloat16)
    wrec_hi = wrec.astype(jnp.bfloat16)
    wrec_lo = (wrec - wrec_hi.astype(jnp.float32)).astype(jnp.bfloat16)

    def mm(a, b, dims):
        return lax.dot_general(a, b, dims,
                               preferred_element_type=jnp.float32)

    dn_in = (((0,), (0,)), ((), ()))
    dn_rec = (((1,), (0,)), ((), ()))

    def step(t, h):
        xt = x_ref[pl.ds(t, 1)][0]
        oht = (lax.broadcasted_iota(jnp.int32, (INPUT_SIZE, B), 0)
               == xt).astype(jnp.bfloat16)
        inc = mm(oht, win_hi, dn_in) + mm(oht, win_lo, dn_in)
        h_hi = h.astype(jnp.bfloat16)
        h_lo = (h - h_hi.astype(jnp.float32)).astype(jnp.bfloat16)
        rec = (mm(h_hi, wrec_hi, dn_rec) + mm(h_hi, wrec_lo, dn_rec)
               + mm(h_lo, wrec_hi, dn_rec))
        pre = bias + inc + rec
        out_ref[pl.ds(t, 1)] = pre[:, N_UNITS - OUTPUT_SIZE:].reshape(
            1, B, OUTPUT_SIZE)
        return jnp.where(mask, pre, jnp.tanh(pre))

    lax.fori_loop(0, T, step, jnp.zeros((B, N_UNITS), jnp.float32))


def kernel(x, tau, logits, grid, codelengths, in_src, in_dst, rec_src,
           rec_dst, out_idx):
    del tau, out_idx
    flat_idx = jnp.concatenate([
        in_src * N_UNITS + in_dst,
        W_REC_OFF + rec_src * N_UNITS + rec_dst,
        BIAS_OFF + jnp.arange(N_UNITS, dtype=jnp.int32),
    ]).reshape(NW, ROWS_W)
    pad = jnp.full((NW, PAD_W - ROWS_W), TRASH_OFF, jnp.int32)
    idx_pad = jnp.concatenate([flat_idx, pad], axis=1).reshape(NW, -1, 128)

    dense_p, ecl_p = _sc_prep(logits.reshape(-1), grid, codelengths, idx_pad)

    d3 = dense_p.reshape(2, DENSE_ROWS, N_UNITS)
    dwin = d3[:, :INPUT_SIZE]
    dwrec = d3[:, INPUT_SIZE:INPUT_SIZE + N_UNITS]
    dbias = d3[:, INPUT_SIZE + N_UNITS:INPUT_SIZE + N_UNITS + 1]

    x3 = x.T.reshape(T, 1, B)
    out_t, ecl = pl.pallas_call(
        _scan_body,
        out_shape=(
            jax.ShapeDtypeStruct((T, B, OUTPUT_SIZE), jnp.float32),
            jax.ShapeDtypeStruct((1, 1), jnp.float32),
        ),
    )(x3, dwin, dwrec, dbias, ecl_p)

    logits_out = jnp.transpose(out_t, (1, 0, 2))
    return logits_out, ecl[0, 0]


# speedup vs baseline: 385.4243x; 1.1318x over previous
import functools

import jax
import jax.numpy as jnp
from jax import lax
from jax.experimental import pallas as pl
from jax.experimental.pallas import tpu as pltpu
from jax.experimental.pallas import tpu_sc as plsc

N_UNITS = 512
INPUT_SIZE = 128
OUTPUT_SIZE = 128
E_IN = 4096
E_REC = 16384
M_GRID = 16
B = 16
T = 512
N_TOTAL = E_IN + E_REC + N_UNITS

NW = 32
ROWS_W = N_TOTAL // NW
BLOCKS_W = ROWS_W // 16
PAD_W = 768
W_REC_OFF = INPUT_SIZE * N_UNITS
BIAS_OFF = W_REC_OFF + N_UNITS * N_UNITS
TRASH_OFF = BIAS_OFF + N_UNITS
DENSE_ROWS = 642
DENSE = DENSE_ROWS * N_UNITS
ZCHUNK = DENSE // 16


def _sc_prep_body(logits_hbm, grid_hbm, cl_hbm, idx_hbm, dense_hbm, ecl_hbm,
                  lbuf, wbuf, idxbuf, gridbuf, clbuf, eclbuf, zbuf, shared):
    cid = lax.axis_index("c")
    sid = lax.axis_index("s")
    w = sid * 2 + cid

    pltpu.sync_copy(logits_hbm.at[pl.ds(w * ROWS_W * M_GRID, ROWS_W * M_GRID)],
                    lbuf)
    pltpu.sync_copy(grid_hbm, gridbuf)
    pltpu.sync_copy(cl_hbm, clbuf)
    pltpu.sync_copy(idx_hbm.at[w], idxbuf)

    def zero_body(i, carry):
        zbuf[pl.ds(i * 16, 16)] = jnp.zeros((16,), jnp.float32)
        return carry
    lax.fori_loop(0, ZCHUNK // 16, zero_body, 0)
    pltpu.sync_copy(zbuf, shared.at[pl.ds(sid * ZCHUNK, ZCHUNK)])

    iota = lax.broadcasted_iota(jnp.int32, (16,), 0)
    row_stride = iota * M_GRID
    clv = clbuf[...]

    def block_body(b, ecl_acc):
        base = b * (16 * M_GRID)
        mv = jnp.full((16,), -jnp.inf, jnp.float32)
        ai = jnp.zeros((16,), jnp.int32)
        cols = []
        for m in range(M_GRID):
            v = plsc.load_gather(lbuf, [base + row_stride + m])
            upd = v > mv
            mv = jnp.where(upd, v, mv)
            ai = jnp.where(upd, m, ai)
            cols.append(v)
        wv = plsc.load_gather(gridbuf, [ai])
        wbuf[pl.ds(b * 16, 16)] = wv
        sv = jnp.zeros((16,), jnp.float32)
        nv = jnp.zeros((16,), jnp.float32)
        for m in range(M_GRID):
            e = jnp.exp(cols[m] - mv)
            sv = sv + e
            nv = nv + clv[m] * e
        return ecl_acc + nv / sv

    ecl_acc = lax.fori_loop(0, BLOCKS_W, block_body,
                            jnp.zeros((16,), jnp.float32))

    for j in range(BLOCKS_W, PAD_W // 16):
        wbuf[pl.ds(j * 16, 16)] = jnp.zeros((16,), jnp.float32)

    plsc.subcore_barrier()
    for j in range(PAD_W // 128):
        pltpu.sync_copy(wbuf.at[pl.ds(j * 128, 128)],
                        shared.at[idxbuf.at[j]], add=True)
    plsc.subcore_barrier()

    @pl.when(sid == 0)
    def _():
        pltpu.sync_copy(shared, dense_hbm.at[cid])

    tot = jnp.sum(ecl_acc)
    eclbuf[...] = jnp.where(iota == 0, tot, 0.0)
    pltpu.sync_copy(eclbuf, ecl_hbm.at[w])


_sc_prep = functools.partial(
    pl.kernel,
    mesh=plsc.VectorSubcoreMesh(core_axis_name="c", subcore_axis_name="s"),
    compiler_params=pltpu.CompilerParams(needs_layout_passes=False),
    out_type=(
        jax.ShapeDtypeStruct((2, DENSE), jnp.float32),
        jax.ShapeDtypeStruct((NW, 16), jnp.float32),
    ),
    scratch_types=[
        pltpu.VMEM((ROWS_W * M_GRID,), jnp.float32),
        pltpu.VMEM((PAD_W,), jnp.float32),
        pltpu.VMEM((PAD_W // 128, 128), jnp.int32),
        pltpu.VMEM((16,), jnp.float32),
        pltpu.VMEM((16,), jnp.float32),
        pltpu.VMEM((16,), jnp.float32),
        pltpu.VMEM((ZCHUNK,), jnp.float32),
        pltpu.VMEM_SHARED((DENSE,), jnp.float32),
    ],
)(_sc_prep_body)


def _scan_body(x4_ref, dwin_ref, dwrec_ref, dbias_ref, eclp_ref,
               out_ref, ecl_ref, binc_ref):
    win = dwin_ref[0] + dwin_ref[1]
    wrec = dwrec_ref[0] + dwrec_ref[1]
    bias = dbias_ref[0] + dbias_ref[1]
    p = jnp.sum(eclp_ref[...], axis=0, keepdims=True)
    ecl_ref[...] = jnp.sum(p, axis=1, keepdims=True)
    mask = lax.broadcasted_iota(jnp.int32, (B, N_UNITS), 1) >= (
        N_UNITS - OUTPUT_SIZE)

    win_hi = win.astype(jnp.bfloat16)
    win_lo = (win - win_hi.astype(jnp.float32)).astype(jnp.bfloat16)
    wrec_hi = wrec.astype(jnp.bfloat16)
    wrec_lo = (wrec - wrec_hi.astype(jnp.float32)).astype(jnp.bfloat16)

    def mm(a, b):
        return lax.dot_general(a, b, (((1,), (0,)), ((), ())),
                               preferred_element_type=jnp.float32)

    ohall = (lax.broadcasted_iota(jnp.int32, (T * B, INPUT_SIZE), 1)
             == x4_ref[...]).astype(jnp.bfloat16)
    binc_ref[...] = bias + mm(ohall, win_hi) + mm(ohall, win_lo)

    def step(t, h):
        h_hi = h.astype(jnp.bfloat16)
        h_lo = (h - h_hi.astype(jnp.float32)).astype(jnp.bfloat16)
        rec = (mm(h_hi, wrec_hi) + mm(h_hi, wrec_lo) + mm(h_lo, wrec_hi))
        pre = binc_ref[pl.ds(t * B, B)] + rec
        out_ref[pl.ds(t * B, B)] = pre[:, N_UNITS - OUTPUT_SIZE:]
        return jnp.where(mask, pre, jnp.tanh(pre))

    lax.fori_loop(0, T, step, jnp.zeros((B, N_UNITS), jnp.float32))


def kernel(x, tau, logits, grid, codelengths, in_src, in_dst, rec_src,
           rec_dst, out_idx):
    del tau, out_idx
    flat_idx = jnp.concatenate([
        in_src * N_UNITS + in_dst,
        W_REC_OFF + rec_src * N_UNITS + rec_dst,
        BIAS_OFF + jnp.arange(N_UNITS, dtype=jnp.int32),
    ]).reshape(NW, ROWS_W)
    pad = jnp.full((NW, PAD_W - ROWS_W), TRASH_OFF, jnp.int32)
    idx_pad = jnp.concatenate([flat_idx, pad], axis=1).reshape(NW, -1, 128)

    dense_p, ecl_p = _sc_prep(logits.reshape(-1), grid, codelengths, idx_pad)

    d3 = dense_p.reshape(2, DENSE_ROWS, N_UNITS)
    dwin = d3[:, :INPUT_SIZE]
    dwrec = d3[:, INPUT_SIZE:INPUT_SIZE + N_UNITS]
    dbias = d3[:, INPUT_SIZE + N_UNITS:INPUT_SIZE + N_UNITS + 1]

    x4 = x.T.reshape(T * B, 1)
    out2, ecl = pl.pallas_call(
        _scan_body,
        out_shape=(
            jax.ShapeDtypeStruct((T * B, OUTPUT_SIZE), jnp.float32),
            jax.ShapeDtypeStruct((1, 1), jnp.float32),
        ),
        scratch_shapes=[pltpu.VMEM((T * B, N_UNITS), jnp.float32)],
    )(x4, dwin, dwrec, dbias, ecl_p)

    logits_out = jnp.transpose(out2.reshape(T, B, OUTPUT_SIZE), (1, 0, 2))
    return logits_out, ecl[0, 0]


# speedup vs baseline: 386.5027x; 1.0028x over previous
import functools

import jax
import jax.numpy as jnp
from jax import lax
from jax.experimental import pallas as pl
from jax.experimental.pallas import tpu as pltpu
from jax.experimental.pallas import tpu_sc as plsc

N_UNITS = 512
INPUT_SIZE = 128
OUTPUT_SIZE = 128
E_IN = 4096
E_REC = 16384
M_GRID = 16
B = 16
T = 512
N_TOTAL = E_IN + E_REC + N_UNITS

NW = 32
ROWS_W = N_TOTAL // NW
BLOCKS_W = ROWS_W // 16
PAD_W = 768
W_REC_OFF = INPUT_SIZE * N_UNITS
BIAS_OFF = W_REC_OFF + N_UNITS * N_UNITS
TRASH_OFF = BIAS_OFF + N_UNITS
DENSE_ROWS = 642
DENSE = DENSE_ROWS * N_UNITS
ZCHUNK = DENSE // 16


def _sc_prep_body(logits_hbm, grid_hbm, cl_hbm, idx_hbm, dense_hbm, ecl_hbm,
                  lbuf, wbuf, idxbuf, gridbuf, clbuf, eclbuf, zbuf, shared):
    cid = lax.axis_index("c")
    sid = lax.axis_index("s")
    w = sid * 2 + cid

    pltpu.sync_copy(logits_hbm.at[pl.ds(w * ROWS_W * M_GRID, ROWS_W * M_GRID)],
                    lbuf)
    pltpu.sync_copy(grid_hbm, gridbuf)
    pltpu.sync_copy(cl_hbm, clbuf)
    pltpu.sync_copy(idx_hbm.at[w], idxbuf)

    def zero_body(i, carry):
        zbuf[pl.ds(i * 16, 16)] = jnp.zeros((16,), jnp.float32)
        return carry
    lax.fori_loop(0, ZCHUNK // 16, zero_body, 0)
    pltpu.sync_copy(zbuf, shared.at[pl.ds(sid * ZCHUNK, ZCHUNK)])

    iota = lax.broadcasted_iota(jnp.int32, (16,), 0)
    row_stride = iota * M_GRID
    clv = clbuf[...]

    def block_body(b, ecl_acc):
        base = b * (16 * M_GRID)
        mv = jnp.full((16,), -jnp.inf, jnp.float32)
        ai = jnp.zeros((16,), jnp.int32)
        cols = []
        for m in range(M_GRID):
            v = plsc.load_gather(lbuf, [base + row_stride + m])
            upd = v > mv
            mv = jnp.where(upd, v, mv)
            ai = jnp.where(upd, m, ai)
            cols.append(v)
        wv = plsc.load_gather(gridbuf, [ai])
        wbuf[pl.ds(b * 16, 16)] = wv
        sv = jnp.zeros((16,), jnp.float32)
        nv = jnp.zeros((16,), jnp.float32)
        for m in range(M_GRID):
            e = jnp.exp(cols[m] - mv)
            sv = sv + e
            nv = nv + clv[m] * e
        return ecl_acc + nv / sv

    ecl_acc = lax.fori_loop(0, BLOCKS_W, block_body,
                            jnp.zeros((16,), jnp.float32))

    for j in range(BLOCKS_W, PAD_W // 16):
        wbuf[pl.ds(j * 16, 16)] = jnp.zeros((16,), jnp.float32)

    plsc.subcore_barrier()
    for j in range(PAD_W // 128):
        pltpu.sync_copy(wbuf.at[pl.ds(j * 128, 128)],
                        shared.at[idxbuf.at[j]], add=True)
    plsc.subcore_barrier()

    @pl.when(sid == 0)
    def _():
        pltpu.sync_copy(shared, dense_hbm.at[cid])

    tot = jnp.sum(ecl_acc)
    eclbuf[...] = jnp.where(iota == 0, tot, 0.0)
    pltpu.sync_copy(eclbuf, ecl_hbm.at[w])


_sc_prep = functools.partial(
    pl.kernel,
    mesh=plsc.VectorSubcoreMesh(core_axis_name="c", subcore_axis_name="s"),
    compiler_params=pltpu.CompilerParams(needs_layout_passes=False),
    out_type=(
        jax.ShapeDtypeStruct((2, DENSE), jnp.float32),
        jax.ShapeDtypeStruct((NW, 16), jnp.float32),
    ),
    scratch_types=[
        pltpu.VMEM((ROWS_W * M_GRID,), jnp.float32),
        pltpu.VMEM((PAD_W,), jnp.float32),
        pltpu.VMEM((PAD_W // 128, 128), jnp.int32),
        pltpu.VMEM((16,), jnp.float32),
        pltpu.VMEM((16,), jnp.float32),
        pltpu.VMEM((16,), jnp.float32),
        pltpu.VMEM((ZCHUNK,), jnp.float32),
        pltpu.VMEM_SHARED((DENSE,), jnp.float32),
    ],
)(_sc_prep_body)


def _scan_body(x4_ref, dwin_ref, dwrec_ref, dbias_ref, eclp_ref,
               out_ref, ecl_ref, binc_ref):
    win = dwin_ref[0] + dwin_ref[1]
    wrec = dwrec_ref[0] + dwrec_ref[1]
    bias = dbias_ref[0] + dbias_ref[1]
    p = jnp.sum(eclp_ref[...], axis=0, keepdims=True)
    ecl_ref[...] = jnp.sum(p, axis=1, keepdims=True)
    mask = lax.broadcasted_iota(jnp.int32, (B, N_UNITS), 1) >= (
        N_UNITS - OUTPUT_SIZE)

    win_hi = win.astype(jnp.bfloat16)
    win_lo = (win - win_hi.astype(jnp.float32)).astype(jnp.bfloat16)
    wrec_hi = wrec.astype(jnp.bfloat16)
    wrec_lo = (wrec - wrec_hi.astype(jnp.float32)).astype(jnp.bfloat16)
    wcat = jnp.concatenate([wrec_lo, wrec_hi], axis=0)

    def mm(a, b):
        return lax.dot_general(a, b, (((1,), (0,)), ((), ())),
                               preferred_element_type=jnp.float32)

    ohall = (lax.broadcasted_iota(jnp.int32, (T * B, INPUT_SIZE), 1)
             == x4_ref[...]).astype(jnp.bfloat16)
    binc_ref[...] = bias + mm(ohall, win_hi) + mm(ohall, win_lo)

    def step(t, h):
        h_hi = h.astype(jnp.bfloat16)
        h_lo = (h - h_hi.astype(jnp.float32)).astype(jnp.bfloat16)
        hcat = jnp.concatenate([h_hi, h_lo], axis=1)
        rec = mm(h_hi, wrec_hi) + mm(hcat, wcat)
        pre = binc_ref[pl.ds(t * B, B)] + rec
        out_ref[pl.ds(t * B, B)] = pre[:, N_UNITS - OUTPUT_SIZE:]
        return jnp.where(mask, pre, jnp.tanh(pre))

    lax.fori_loop(0, T, step, jnp.zeros((B, N_UNITS), jnp.float32))


def kernel(x, tau, logits, grid, codelengths, in_src, in_dst, rec_src,
           rec_dst, out_idx):
    del tau, out_idx
    flat_idx = jnp.concatenate([
        in_src * N_UNITS + in_dst,
        W_REC_OFF + rec_src * N_UNITS + rec_dst,
        BIAS_OFF + jnp.arange(N_UNITS, dtype=jnp.int32),
    ]).reshape(NW, ROWS_W)
    pad = jnp.full((NW, PAD_W - ROWS_W), TRASH_OFF, jnp.int32)
    idx_pad = jnp.concatenate([flat_idx, pad], axis=1).reshape(NW, -1, 128)

    dense_p, ecl_p = _sc_prep(logits.reshape(-1), grid, codelengths, idx_pad)

    d3 = dense_p.reshape(2, DENSE_ROWS, N_UNITS)
    dwin = d3[:, :INPUT_SIZE]
    dwrec = d3[:, INPUT_SIZE:INPUT_SIZE + N_UNITS]
    dbias = d3[:, INPUT_SIZE + N_UNITS:INPUT_SIZE + N_UNITS + 1]

    x4 = x.T.reshape(T * B, 1)
    out2, ecl = pl.pallas_call(
        _scan_body,
        out_shape=(
            jax.ShapeDtypeStruct((T * B, OUTPUT_SIZE), jnp.float32),
            jax.ShapeDtypeStruct((1, 1), jnp.float32),
        ),
        scratch_shapes=[pltpu.VMEM((T * B, N_UNITS), jnp.float32)],
    )(x4, dwin, dwrec, dbias, ecl_p)

    logits_out = jnp.transpose(out2.reshape(T, B, OUTPUT_SIZE), (1, 0, 2))
    return logits_out, ecl[0, 0]
